# Initial kernel scaffold; baseline (speedup 1.0000x reference)
import jax
import jax.numpy as jnp
from jax.experimental import pallas as pl


def kernel(radiant_heros, dire_heros, E_r, E_d, W1, b1, W2, b2, W3, b3):
    raise NotImplementedError("write your pallas kernel here")


# speedup vs baseline: 5.8614x; 5.8614x over previous
import functools

import jax
import jax.numpy as jnp
from jax import lax
from jax.experimental import pallas as pl
from jax.experimental.pallas import tpu as pltpu
from jax.experimental.pallas import tpu_sc as plsc

VOCAB = 150
EMBED = 32
BATCH = 16384
HIST = 5
PADV = 152


def _prep_body(er_ref, ed_ref, w1_ref, t_ref):
    w1a = w1_ref[0:EMBED, :]
    w1b = w1_ref[EMBED : 2 * EMBED, :]
    t_ref[0:PADV, :] = jnp.dot(er_ref[...], w1a, preferred_element_type=jnp.float32, precision=lax.Precision.HIGHEST)
    t_ref[PADV : 2 * PADV, :] = jnp.dot(ed_ref[...], w1b, preferred_element_type=jnp.float32, precision=lax.Precision.HIGHEST)


def _prep_tables(E_r, E_d, W1):
    er = jnp.pad(E_r, ((0, PADV - VOCAB), (0, 0)))
    ed = jnp.pad(E_d, ((0, PADV - VOCAB), (0, 0)))
    return pl.pallas_call(
        _prep_body,
        out_shape=jax.ShapeDtypeStruct((2 * PADV, EMBED), jnp.float32),
    )(er, ed, W1)


_NC, _NS, _L = 2, 16, 16
_NW = _NC * _NS
_BW = BATCH // _NW
_NG = _BW // _L


def _sc_body(r_hbm, d_hbm, t_hbm, out_hbm, r_v, d_v, t_v, acc_v):
    wid = lax.axis_index("s") * _NC + lax.axis_index("c")
    base = wid * _BW
    pltpu.sync_copy(t_hbm, t_v)
    pltpu.sync_copy(r_hbm.at[:, pl.ds(base, _BW)], r_v)
    pltpu.sync_copy(d_hbm.at[:, pl.ds(base, _BW)], d_v)

    def group(g, carry):
        addr = []
        for h in range(HIST):
            addr.append(r_v[h, pl.ds(g * _L, _L)] * EMBED)
        for h in range(HIST):
            addr.append(d_v[h, pl.ds(g * _L, _L)] * EMBED + PADV * EMBED)
        for j in range(EMBED):
            acc = plsc.load_gather(t_v, [addr[0] + j])
            for h in range(1, 2 * HIST):
                acc = acc + plsc.load_gather(t_v, [addr[h] + j])
            acc_v[j, pl.ds(g * _L, _L)] = acc
        return carry

    lax.fori_loop(0, _NG, group, 0, unroll=False)
    pltpu.sync_copy(acc_v, out_hbm.at[:, pl.ds(base, _BW)])


def _sc_gather(radiant_t, dire_t, table_flat):
    mesh = plsc.VectorSubcoreMesh(core_axis_name="c", subcore_axis_name="s")
    return pl.kernel(
        _sc_body,
        out_type=jax.ShapeDtypeStruct((EMBED, BATCH), jnp.float32),
        mesh=mesh,
        compiler_params=pltpu.CompilerParams(needs_layout_passes=False),
        scratch_types=[
            pltpu.VMEM((HIST, _BW), jnp.int32),
            pltpu.VMEM((HIST, _BW), jnp.int32),
            pltpu.VMEM((2 * PADV * EMBED,), jnp.float32),
            pltpu.VMEM((EMBED, _BW), jnp.float32),
        ],
    )(radiant_t, dire_t, table_flat)


_BN = 4096


def _mlp_body(s_ref, b1_ref, w2_ref, b2_ref, w3_ref, b3_ref, out_ref):
    h1 = jnp.maximum(s_ref[...] + b1_ref[...], 0.0)
    h2 = lax.dot_general(
        w2_ref[...], h1, (((0,), (0,)), ((), ())), preferred_element_type=jnp.float32, precision=lax.Precision.HIGHEST
    )
    h2 = jnp.maximum(h2 + b2_ref[...], 0.0)
    h3 = lax.dot_general(
        w3_ref[...], h2, (((0,), (0,)), ((), ())), preferred_element_type=jnp.float32, precision=lax.Precision.HIGHEST
    )
    out_ref[...] = jnp.maximum(h3 + b3_ref[...], 0.0)


def _mlp(s, b1, W2, b2, W3, b3):
    grid = (BATCH // _BN,)
    return pl.pallas_call(
        _mlp_body,
        grid=grid,
        in_specs=[
            pl.BlockSpec((EMBED, _BN), lambda i: (0, i)),
            pl.BlockSpec((EMBED, 1), lambda i: (0, 0)),
            pl.BlockSpec(W2.shape, lambda i: (0, 0)),
            pl.BlockSpec((EMBED // 2, 1), lambda i: (0, 0)),
            pl.BlockSpec(W3.shape, lambda i: (0, 0)),
            pl.BlockSpec((1, 1), lambda i: (0, 0)),
        ],
        out_specs=pl.BlockSpec((1, _BN), lambda i: (0, i)),
        out_shape=jax.ShapeDtypeStruct((1, BATCH), jnp.float32),
    )(s, b1.reshape(EMBED, 1), W2, b2.reshape(EMBED // 2, 1), W3, b3.reshape(1, 1))


def kernel(radiant_heros, dire_heros, E_r, E_d, W1, b1, W2, b2, W3, b3):
    table = _prep_tables(E_r, E_d, W1)
    s = _sc_gather(radiant_heros.T, dire_heros.T, table.reshape(-1))
    out = _mlp(s, b1, W2, b2, W3, b3)
    return out.reshape(BATCH, 1)


# speedup vs baseline: 14.9559x; 2.5516x over previous
import functools

import jax
import jax.numpy as jnp
from jax import lax
from jax.experimental import pallas as pl
from jax.experimental.pallas import tpu as pltpu
from jax.experimental.pallas import tpu_sc as plsc

VOCAB = 150
EMBED = 32
BATCH = 16384
HIST = 5
PADV = 152
_STRIDE = EMBED + 1


def _prep_body(er_ref, ed_ref, w1_ref, t_ref):
    w1a = w1_ref[0:EMBED, :]
    w1b = w1_ref[EMBED : 2 * EMBED, :]
    t_ref[0:PADV, :] = jnp.dot(er_ref[...], w1a, preferred_element_type=jnp.float32, precision=lax.Precision.HIGHEST)
    t_ref[PADV : 2 * PADV, :] = jnp.dot(ed_ref[...], w1b, preferred_element_type=jnp.float32, precision=lax.Precision.HIGHEST)


def _prep_tables(E_r, E_d, W1):
    er = jnp.pad(E_r, ((0, PADV - VOCAB), (0, 0)))
    ed = jnp.pad(E_d, ((0, PADV - VOCAB), (0, 0)))
    return pl.pallas_call(
        _prep_body,
        out_shape=jax.ShapeDtypeStruct((2 * PADV, EMBED), jnp.float32),
    )(er, ed, W1)


_NC, _NS, _L = 2, 16, 16
_NW = _NC * _NS
_BW = BATCH // _NW
_NG = _BW // _L


def _sc_body(r_hbm, d_hbm, t_hbm, out_hbm, r_v, d_v, t_v, acc_v):
    wid = lax.axis_index("s") * _NC + lax.axis_index("c")
    base = wid * _BW
    pltpu.sync_copy(t_hbm, t_v)
    pltpu.sync_copy(r_hbm.at[:, pl.ds(base, _BW)], r_v)
    pltpu.sync_copy(d_hbm.at[:, pl.ds(base, _BW)], d_v)

    def group(g, carry):
        addr = []
        for h in range(HIST):
            addr.append(r_v[h, pl.ds(g * _L, _L)] * _STRIDE)
        for h in range(HIST):
            addr.append(d_v[h, pl.ds(g * _L, _L)] * _STRIDE + PADV * _STRIDE)
        for j in range(EMBED):
            vals = [plsc.load_gather(t_v, [addr[h] + j]) for h in range(2 * HIST)]
            while len(vals) > 1:
                vals = [a + b for a, b in zip(vals[::2], vals[1::2])] + (
                    [vals[-1]] if len(vals) % 2 else []
                )
            acc_v[j, pl.ds(g * _L, _L)] = vals[0]
        return carry

    lax.fori_loop(0, _NG, group, 0, unroll=False)
    pltpu.sync_copy(acc_v, out_hbm.at[:, pl.ds(base, _BW)])


def _sc_gather(radiant_t, dire_t, table_flat):
    mesh = plsc.VectorSubcoreMesh(core_axis_name="c", subcore_axis_name="s")
    return pl.kernel(
        _sc_body,
        out_type=jax.ShapeDtypeStruct((EMBED, BATCH), jnp.float32),
        mesh=mesh,
        compiler_params=pltpu.CompilerParams(needs_layout_passes=False),
        scratch_types=[
            pltpu.VMEM((HIST, _BW), jnp.int32),
            pltpu.VMEM((HIST, _BW), jnp.int32),
            pltpu.VMEM((2 * PADV * _STRIDE,), jnp.float32),
            pltpu.VMEM((EMBED, _BW), jnp.float32),
        ],
    )(radiant_t, dire_t, table_flat)


_BN = 4096


def _mlp_body(s_ref, b1_ref, w2_ref, b2_ref, w3_ref, b3_ref, out_ref):
    h1 = jnp.maximum(s_ref[...] + b1_ref[...], 0.0)
    h2 = lax.dot_general(
        w2_ref[...], h1, (((0,), (0,)), ((), ())), preferred_element_type=jnp.float32, precision=lax.Precision.HIGHEST
    )
    h2 = jnp.maximum(h2 + b2_ref[...], 0.0)
    h3 = lax.dot_general(
        w3_ref[...], h2, (((0,), (0,)), ((), ())), preferred_element_type=jnp.float32, precision=lax.Precision.HIGHEST
    )
    out_ref[...] = jnp.maximum(h3 + b3_ref[...], 0.0)


def _mlp(s, b1, W2, b2, W3, b3):
    grid = (BATCH // _BN,)
    return pl.pallas_call(
        _mlp_body,
        grid=grid,
        in_specs=[
            pl.BlockSpec((EMBED, _BN), lambda i: (0, i)),
            pl.BlockSpec((EMBED, 1), lambda i: (0, 0)),
            pl.BlockSpec(W2.shape, lambda i: (0, 0)),
            pl.BlockSpec((EMBED // 2, 1), lambda i: (0, 0)),
            pl.BlockSpec(W3.shape, lambda i: (0, 0)),
            pl.BlockSpec((1, 1), lambda i: (0, 0)),
        ],
        out_specs=pl.BlockSpec((1, _BN), lambda i: (0, i)),
        out_shape=jax.ShapeDtypeStruct((1, BATCH), jnp.float32),
    )(s, b1.reshape(EMBED, 1), W2, b2.reshape(EMBED // 2, 1), W3, b3.reshape(1, 1))


def kernel(radiant_heros, dire_heros, E_r, E_d, W1, b1, W2, b2, W3, b3):
    table = _prep_tables(E_r, E_d, W1)
    table = jnp.pad(table, ((0, 0), (0, _STRIDE - EMBED))).reshape(-1)
    s = _sc_gather(radiant_heros.T, dire_heros.T, table)
    out = _mlp(s, b1, W2, b2, W3, b3)
    return out.reshape(BATCH, 1)


# speedup vs baseline: 17.4205x; 1.1648x over previous
import jax
import jax.numpy as jnp
from jax import lax
from jax.experimental import pallas as pl
from jax.experimental.pallas import tpu as pltpu
from jax.experimental.pallas import tpu_sc as plsc

VOCAB = 150
EMBED = 32
BATCH = 16384
HIST = 5
PADV = 152
_STRIDE = EMBED + 1

_NC, _NS, _L = 2, 16, 16
_NW = _NC * _NS
_BW = BATCH // _NW
_NG = _BW // _L


def _tree_sum(vals):
    while len(vals) > 1:
        vals = [a + b for a, b in zip(vals[::2], vals[1::2])] + (
            [vals[-1]] if len(vals) % 2 else []
        )
    return vals[0]


def _sc_body(r_hbm, d_hbm, t_hbm, out_hbm, r_v, d_v, t_v, acc_v):
    wid = lax.axis_index("s") * _NC + lax.axis_index("c")
    base = wid * _BW
    pltpu.sync_copy(t_hbm, t_v)
    pltpu.sync_copy(r_hbm.at[:, pl.ds(base, _BW)], r_v)
    pltpu.sync_copy(d_hbm.at[:, pl.ds(base, _BW)], d_v)

    def group(g, carry):
        ra = [r_v[h, pl.ds(g * _L, _L)] * _STRIDE for h in range(HIST)]
        da = [
            d_v[h, pl.ds(g * _L, _L)] * _STRIDE + PADV * _STRIDE
            for h in range(HIST)
        ]
        for j in range(EMBED):
            rs = _tree_sum([plsc.load_gather(t_v, [a + j]) for a in ra])
            ds = _tree_sum([plsc.load_gather(t_v, [a + j]) for a in da])
            acc_v[j, pl.ds(g * _L, _L)] = rs
            acc_v[EMBED + j, pl.ds(g * _L, _L)] = ds
        return carry

    lax.fori_loop(0, _NG, group, 0, unroll=False)
    pltpu.sync_copy(acc_v, out_hbm.at[:, pl.ds(base, _BW)])


def _sc_gather(radiant_t, dire_t, table_flat):
    mesh = plsc.VectorSubcoreMesh(core_axis_name="c", subcore_axis_name="s")
    return pl.kernel(
        _sc_body,
        out_type=jax.ShapeDtypeStruct((2 * EMBED, BATCH), jnp.float32),
        mesh=mesh,
        compiler_params=pltpu.CompilerParams(needs_layout_passes=False),
        scratch_types=[
            pltpu.VMEM((HIST, _BW), jnp.int32),
            pltpu.VMEM((HIST, _BW), jnp.int32),
            pltpu.VMEM((2 * PADV * _STRIDE,), jnp.float32),
            pltpu.VMEM((2 * EMBED, _BW), jnp.float32),
        ],
    )(radiant_t, dire_t, table_flat)


_BN = 4096


def _mlp_body(x_ref, w1_ref, b1_ref, w2_ref, b2_ref, w3_ref, b3_ref, out_ref):
    h1 = lax.dot_general(
        w1_ref[...], x_ref[...], (((0,), (0,)), ((), ())),
        preferred_element_type=jnp.float32,
    )
    h1 = jnp.maximum(h1 + b1_ref[...], 0.0)
    h2 = lax.dot_general(
        w2_ref[...], h1, (((0,), (0,)), ((), ())),
        preferred_element_type=jnp.float32,
    )
    h2 = jnp.maximum(h2 + b2_ref[...], 0.0)
    h3 = lax.dot_general(
        w3_ref[...], h2, (((0,), (0,)), ((), ())),
        preferred_element_type=jnp.float32,
    )
    out_ref[...] = jnp.maximum(h3 + b3_ref[...], 0.0)


def _mlp(x, W1, b1, W2, b2, W3, b3):
    grid = (BATCH // _BN,)
    return pl.pallas_call(
        _mlp_body,
        grid=grid,
        in_specs=[
            pl.BlockSpec((2 * EMBED, _BN), lambda i: (0, i)),
            pl.BlockSpec(W1.shape, lambda i: (0, 0)),
            pl.BlockSpec((EMBED, 1), lambda i: (0, 0)),
            pl.BlockSpec(W2.shape, lambda i: (0, 0)),
            pl.BlockSpec((EMBED // 2, 1), lambda i: (0, 0)),
            pl.BlockSpec(W3.shape, lambda i: (0, 0)),
            pl.BlockSpec((1, 1), lambda i: (0, 0)),
        ],
        out_specs=pl.BlockSpec((1, _BN), lambda i: (0, i)),
        out_shape=jax.ShapeDtypeStruct((1, BATCH), jnp.float32),
    )(
        x, W1, b1.reshape(EMBED, 1), W2, b2.reshape(EMBED // 2, 1), W3,
        b3.reshape(1, 1),
    )


def kernel(radiant_heros, dire_heros, E_r, E_d, W1, b1, W2, b2, W3, b3):
    pad = ((0, PADV - VOCAB), (0, _STRIDE - EMBED))
    table = jnp.concatenate([jnp.pad(E_r, pad), jnp.pad(E_d, pad)]).reshape(-1)
    x = _sc_gather(radiant_heros.T, dire_heros.T, table)
    out = _mlp(x, W1, b1, W2, b2, W3, b3)
    return out.reshape(BATCH, 1)
